# baseline (speedup 1.0000x reference)
import functools

import jax
import jax.numpy as jnp
from jax import lax
from jax.experimental import pallas as pl
from jax.experimental.pallas import tpu as pltpu
from jax.experimental.pallas import tpu_sc as plsc

B = 16384
NC, NS = 2, 16
NW = NC * NS
BPW = B // NW
TPW = 3 * BPW * 2
GPW = BPW * 2

SELU_ALPHA = 1.6732632423543772
SELU_SCALE = 1.0507009873554805


@functools.cache
def _sc_gather():
    mesh = plsc.VectorSubcoreMesh(core_axis_name="c", subcore_axis_name="s")
    out_t = jax.ShapeDtypeStruct((NW, BPW * 2), jnp.float32)

    @functools.partial(
        pl.kernel,
        mesh=mesh,
        out_type=(out_t, out_t, out_t, out_t),
        compiler_params=pltpu.CompilerParams(use_tc_tiling_on_sc=False),
        scratch_types=[
            pltpu.VMEM((TPW,), jnp.int32),
            pltpu.VMEM((GPW,), jnp.int32),
            pltpu.VMEM((TPW,), jnp.float32),
            pltpu.VMEM((GPW,), jnp.float32),
            pltpu.SemaphoreType.DMA,
        ],
    )
    def body(tidx_hbm, gidx_hbm, team_hbm, ground_hbm,
             o1, o2, o3, o4, ti_v, gi_v, tv_v, gv_v, sem):
        wid = lax.axis_index("s") * NC + lax.axis_index("c")
        pltpu.sync_copy(tidx_hbm.at[wid], ti_v)
        pltpu.sync_copy(gidx_hbm.at[wid], gi_v)
        ct = pltpu.async_copy(team_hbm.at[ti_v], tv_v, sem)
        cg = pltpu.async_copy(ground_hbm.at[gi_v], gv_v, sem)
        ct.wait()
        cg.wait()
        for j, o in enumerate((o1, o2, o3)):
            pltpu.sync_copy(tv_v.at[pl.ds(j * BPW * 2, BPW * 2)], o.at[wid])
        pltpu.sync_copy(gv_v, o4.at[wid])

    return body


def _mlp_body(t1, t2, tw, hg, toss, te, w1, b1, w2, b2, out):
    tf = toss[...].astype(jnp.float32)
    onehot = jnp.concatenate([1.0 - tf, tf], axis=1)
    tv = jnp.dot(onehot, te[...], preferred_element_type=jnp.float32)
    x = jnp.concatenate([t1[...], t2[...], tw[...], hg[...], tv], axis=1)
    h = jnp.dot(x, w1[...], preferred_element_type=jnp.float32) + b1[...]
    a = SELU_SCALE * jnp.where(h > 0, h, SELU_ALPHA * (jnp.exp(jnp.minimum(h, 0.0)) - 1.0))
    logits = jnp.dot(a, w2[...], preferred_element_type=jnp.float32) + b2[...]
    s = logits - jnp.max(logits, axis=1, keepdims=True)
    out[...] = s - jnp.log(jnp.sum(jnp.exp(s), axis=1, keepdims=True))


@functools.cache
def _mlp(gb: int):
    full = lambda shape: pl.BlockSpec(shape, lambda i: (0, 0))
    blk = lambda shape: pl.BlockSpec(shape, lambda i: (i, 0))
    return pl.pallas_call(
        _mlp_body,
        grid=(B // gb,),
        in_specs=[
            blk((gb, 2)), blk((gb, 2)), blk((gb, 2)), blk((gb, 2)),
            blk((gb, 1)),
            full((2, 1)), full((9, 20)), full((1, 20)), full((20, 2)), full((1, 2)),
        ],
        out_specs=blk((gb, 2)),
        out_shape=jax.ShapeDtypeStruct((B, 2), jnp.float32),
    )


def kernel(batch, team_emb, ground_emb, toss_emb, W1, b1, W2, b2):
    ti = batch[:, :3].T * 2
    ti = jnp.stack([ti, ti + 1], axis=-1)
    ti = ti.reshape(3, NW, BPW * 2).transpose(1, 0, 2).reshape(NW, TPW)
    gi = batch[:, 3] * 2
    gi = jnp.stack([gi, gi + 1], axis=-1).reshape(NW, GPW)

    t1, t2, tw, hg = _sc_gather()(
        ti, gi, team_emb.reshape(-1), ground_emb.reshape(-1))

    toss = batch[:, 4:5]
    return _mlp(2048)(
        t1.reshape(B, 2), t2.reshape(B, 2), tw.reshape(B, 2), hg.reshape(B, 2),
        toss, toss_emb, W1, b1.reshape(1, -1), W2, b2.reshape(1, -1),
    )


# speedup vs baseline: 9.0184x; 9.0184x over previous
import functools

import jax
import jax.numpy as jnp
from jax import lax
from jax.experimental import pallas as pl
from jax.experimental.pallas import tpu as pltpu
from jax.experimental.pallas import tpu_sc as plsc

B = 16384
TEAM_N = 1000000
GROUND_N = 100000
NC, NS = 2, 16
NW = NC * NS
BPW = B // NW
TPW = 3 * BPW * 2
GPW = BPW * 2

SELU_ALPHA = 1.6732632423543772
SELU_SCALE = 1.0507009873554805


@functools.cache
def _sc_gather():
    mesh = plsc.VectorSubcoreMesh(core_axis_name="c", subcore_axis_name="s")
    out_t = jax.ShapeDtypeStruct((NW, BPW * 2), jnp.float32)

    @functools.partial(
        pl.kernel,
        mesh=mesh,
        out_type=(out_t, out_t, out_t, out_t),
        compiler_params=pltpu.CompilerParams(use_tc_tiling_on_sc=False),
        scratch_types=[
            pltpu.VMEM((TPW,), jnp.int32),
            pltpu.VMEM((GPW,), jnp.int32),
            pltpu.VMEM((TPW,), jnp.float32),
            pltpu.VMEM((GPW,), jnp.float32),
            pltpu.SemaphoreType.DMA,
        ],
    )
    def body(tidx_hbm, gidx_hbm, team_hbm, ground_hbm,
             o1, o2, o3, o4, ti_v, gi_v, tv_v, gv_v, sem):
        wid = lax.axis_index("s") * NC + lax.axis_index("c")
        pltpu.sync_copy(tidx_hbm.at[wid], ti_v)
        pltpu.sync_copy(gidx_hbm.at[wid], gi_v)
        ct = pltpu.async_copy(team_hbm.at[ti_v], tv_v, sem)
        cg = pltpu.async_copy(ground_hbm.at[gi_v], gv_v, sem)
        ct.wait()
        cg.wait()
        for j, o in enumerate((o1, o2, o3)):
            pltpu.sync_copy(tv_v.at[pl.ds(j * BPW * 2, BPW * 2)], o.at[wid])
        pltpu.sync_copy(gv_v, o4.at[wid])

    return body


def _mlp_body(t1, t2, tw, hg, toss, te, w1, b1, w2, b2, out):
    tf = toss[...].astype(jnp.float32)
    onehot = jnp.concatenate([1.0 - tf, tf], axis=1)
    tv = jnp.dot(onehot, te[...], preferred_element_type=jnp.float32)
    x = jnp.concatenate([t1[...], t2[...], tw[...], hg[...], tv], axis=1)
    h = jnp.dot(x, w1[...], preferred_element_type=jnp.float32) + b1[...]
    a = SELU_SCALE * jnp.where(h > 0, h, SELU_ALPHA * (jnp.exp(jnp.minimum(h, 0.0)) - 1.0))
    logits = jnp.dot(a, w2[...], preferred_element_type=jnp.float32) + b2[...]
    s = logits - jnp.max(logits, axis=1, keepdims=True)
    out[...] = s - jnp.log(jnp.sum(jnp.exp(s), axis=1, keepdims=True))


@functools.cache
def _mlp(gb: int):
    full = lambda shape: pl.BlockSpec(shape, lambda i: (0, 0))
    blk = lambda shape: pl.BlockSpec(shape, lambda i: (i, 0))
    return pl.pallas_call(
        _mlp_body,
        grid=(B // gb,),
        in_specs=[
            blk((gb, 2)), blk((gb, 2)), blk((gb, 2)), blk((gb, 2)),
            blk((gb, 1)),
            full((2, 1)), full((9, 20)), full((1, 20)), full((20, 2)), full((1, 2)),
        ],
        out_specs=blk((gb, 2)),
        out_shape=jax.ShapeDtypeStruct((B, 2), jnp.float32),
    )


def kernel(batch, team_emb, ground_emb, toss_emb, W1, b1, W2, b2):
    ti = batch[:, :3].T
    ti = jnp.stack([ti, ti + TEAM_N], axis=-1)
    ti = ti.reshape(3, NW, BPW * 2).transpose(1, 0, 2).reshape(NW, TPW)
    gi = batch[:, 3]
    gi = jnp.stack([gi, gi + GROUND_N], axis=-1).reshape(NW, GPW)

    t1, t2, tw, hg = _sc_gather()(
        ti, gi, team_emb.T.reshape(-1), ground_emb.T.reshape(-1))

    toss = batch[:, 4:5]
    return _mlp(2048)(
        t1.reshape(B, 2), t2.reshape(B, 2), tw.reshape(B, 2), hg.reshape(B, 2),
        toss, toss_emb, W1, b1.reshape(1, -1), W2, b2.reshape(1, -1),
    )


# speedup vs baseline: 11.5901x; 1.2852x over previous
import functools

import jax
import jax.numpy as jnp
from jax import lax
from jax.experimental import pallas as pl
from jax.experimental.pallas import tpu as pltpu
from jax.experimental.pallas import tpu_sc as plsc

B = 16384
TEAM_N = 1000000
GROUND_N = 100000
NC, NS = 2, 16
NW = NC * NS
BPW = B // NW
NPLANES = 9

SELU_ALPHA = 1.6732632423543772
SELU_SCALE = 1.0507009873554805


@functools.cache
def _sc_gather():
    mesh = plsc.VectorSubcoreMesh(core_axis_name="c", subcore_axis_name="s")

    @functools.partial(
        pl.kernel,
        mesh=mesh,
        out_type=jax.ShapeDtypeStruct((NPLANES, NW, BPW), jnp.float32),
        compiler_params=pltpu.CompilerParams(use_tc_tiling_on_sc=False),
        scratch_types=[
            pltpu.VMEM((NPLANES * BPW,), jnp.int32),
            pltpu.VMEM((NPLANES * BPW,), jnp.float32),
            pltpu.SemaphoreType.DMA,
        ],
    )
    def body(idx_hbm, team_hbm, ground_hbm, toss_hbm, o, idx_v, val_v, sem):
        wid = lax.axis_index("s") * NC + lax.axis_index("c")
        sl = lambda j: pl.ds(j * BPW, BPW)
        cps = [pltpu.async_copy(idx_hbm.at[j, wid], idx_v.at[sl(j)], sem)
               for j in range(NPLANES)]
        for cp in cps:
            cp.wait()
        gs = []
        for j in range(NPLANES):
            src = team_hbm if j < 6 else (ground_hbm if j < 8 else toss_hbm)
            gs.append(pltpu.async_copy(src.at[idx_v.at[sl(j)]],
                                       val_v.at[sl(j)], sem))
        for g in gs:
            g.wait()
        ws = [pltpu.async_copy(val_v.at[sl(j)], o.at[j, wid], sem)
              for j in range(NPLANES)]
        for w in ws:
            w.wait()

    return body


_FEAT_SRC = (0, 3, 1, 4, 2, 5, 6, 7, 8)


def _mlp_body(g, w1, b1, w2, b2, out):
    xs = [g[j] for j in _FEAT_SRC]
    acts = []
    for k in range(20):
        h = b1[0, k]
        for j in range(9):
            h = h + xs[j] * w1[j, k]
        acts.append(SELU_SCALE * jnp.where(
            h > 0, h, SELU_ALPHA * (jnp.exp(jnp.minimum(h, 0.0)) - 1.0)))
    outs = []
    for c in range(2):
        l = b2[0, c]
        for k in range(20):
            l = l + acts[k] * w2[k, c]
        outs.append(l)
    m = jnp.maximum(outs[0], outs[1])
    s0, s1 = outs[0] - m, outs[1] - m
    lse = jnp.log(jnp.exp(s0) + jnp.exp(s1))
    out[0] = s0 - lse
    out[1] = s1 - lse


@functools.cache
def _mlp():
    return pl.pallas_call(
        _mlp_body,
        out_shape=jax.ShapeDtypeStruct((2, 128, 128), jnp.float32),
    )


def kernel(batch, team_emb, ground_emb, toss_emb, W1, b1, W2, b2):
    t0, t1, t2, gr, toss = (batch[:, j] for j in range(5))
    idx9 = jnp.stack([t0, t1, t2,
                      t0 + TEAM_N, t1 + TEAM_N, t2 + TEAM_N,
                      gr, gr + GROUND_N, toss]).reshape(NPLANES, NW, BPW)

    gath = _sc_gather()(idx9, team_emb.T.reshape(-1),
                        ground_emb.T.reshape(-1), toss_emb.reshape(-1))

    o2 = _mlp()(gath.reshape(NPLANES, 128, 128),
                W1, b1.reshape(1, -1), W2, b2.reshape(1, -1))
    return o2.reshape(2, B).T


# speedup vs baseline: 31.1224x; 2.6853x over previous
import functools

import jax
import jax.numpy as jnp
from jax import lax
from jax.experimental import pallas as pl
from jax.experimental.pallas import tpu as pltpu
from jax.experimental.pallas import tpu_sc as plsc

B = 16384
TEAM_N = 1000000
GROUND_N = 100000
NC, NS = 2, 16
NW = NC * NS
BPW = B // NW
NPLANES = 8

SELU_ALPHA = 1.6732632423543772
SELU_SCALE = 1.0507009873554805


@functools.cache
def _sc_gather():
    mesh = plsc.VectorSubcoreMesh(core_axis_name="c", subcore_axis_name="s")

    @functools.partial(
        pl.kernel,
        mesh=mesh,
        out_type=jax.ShapeDtypeStruct((NPLANES, NW, BPW), jnp.float32),
        compiler_params=pltpu.CompilerParams(use_tc_tiling_on_sc=False),
        scratch_types=[
            pltpu.VMEM((NPLANES * BPW,), jnp.int32),
            pltpu.VMEM((NPLANES * BPW,), jnp.float32),
            pltpu.SemaphoreType.DMA,
        ],
    )
    def body(idx_hbm, team_hbm, ground_hbm, o, idx_v, val_v, sem):
        wid = lax.axis_index("s") * NC + lax.axis_index("c")
        sl = lambda j: pl.ds(j * BPW, BPW)
        cps = [pltpu.async_copy(idx_hbm.at[j, wid], idx_v.at[sl(j)], sem)
               for j in range(NPLANES)]
        for cp in cps:
            cp.wait()
        gs = []
        for j in range(NPLANES):
            src = team_hbm if j < 6 else ground_hbm
            gs.append(pltpu.async_copy(src.at[idx_v.at[sl(j)]],
                                       val_v.at[sl(j)], sem))
        for g in gs:
            g.wait()
        ws = [pltpu.async_copy(val_v.at[sl(j)], o.at[j, wid], sem)
              for j in range(NPLANES)]
        for w in ws:
            w.wait()

    return body


_FEAT_SRC = (0, 3, 1, 4, 2, 5, 6, 7)


def _mlp_body(g, toss, te, w1, b1, w2, b2, out):
    xs = [g[j] for j in _FEAT_SRC]
    tf = toss[...].astype(jnp.float32)
    te0, te1 = te[0, 0], te[1, 0]
    xs.append(te0 + (te1 - te0) * tf)
    acts = []
    for k in range(20):
        h = b1[0, k]
        for j in range(9):
            h = h + xs[j] * w1[j, k]
        acts.append(SELU_SCALE * jnp.where(
            h > 0, h, SELU_ALPHA * (jnp.exp(jnp.minimum(h, 0.0)) - 1.0)))
    outs = []
    for c in range(2):
        l = b2[0, c]
        for k in range(20):
            l = l + acts[k] * w2[k, c]
        outs.append(l)
    m = jnp.maximum(outs[0], outs[1])
    s0, s1 = outs[0] - m, outs[1] - m
    lse = jnp.log(jnp.exp(s0) + jnp.exp(s1))
    out[0] = s0 - lse
    out[1] = s1 - lse


@functools.cache
def _mlp():
    return pl.pallas_call(
        _mlp_body,
        out_shape=jax.ShapeDtypeStruct((2, 128, 128), jnp.float32),
    )


def kernel(batch, team_emb, ground_emb, toss_emb, W1, b1, W2, b2):
    t0, t1, t2, gr = (batch[:, j] for j in range(4))
    idx8 = jnp.stack([t0, t1, t2,
                      t0 + TEAM_N, t1 + TEAM_N, t2 + TEAM_N,
                      gr, gr + GROUND_N]).reshape(NPLANES, NW, BPW)

    gath = _sc_gather()(idx8, team_emb.T.reshape(-1), ground_emb.T.reshape(-1))

    o2 = _mlp()(gath.reshape(NPLANES, 128, 128),
                batch[:, 4].reshape(128, 128), toss_emb,
                W1, b1.reshape(1, -1), W2, b2.reshape(1, -1))
    return o2.reshape(2, B).T
